# baseline (speedup 1.0000x reference)
import functools

import jax
import jax.numpy as jnp
import numpy as np
from jax import lax
from jax.experimental import pallas as pl
from jax.experimental.pallas import tpu as pltpu
from jax.experimental.pallas import tpu_sc as plsc

N_PTS = 2_000_000
RES = 256
SENTINEL = RES * RES * RES
LO = np.float32(0.0) + np.float32(1e-5)
HI = np.float32(1.0) - np.float32(1e-5)

NC, NS, L = 2, 16, 16
NW = NC * NS

N_SEG = 4
SEG = N_PTS // N_SEG

C = 2000
N_CHUNKS = SEG // C
GROUPS = C // L

_mesh = plsc.VectorSubcoreMesh(core_axis_name="c", subcore_axis_name="s")


@functools.partial(
    pl.kernel,
    out_type=jax.ShapeDtypeStruct((SEG,), jnp.bool_),
    mesh=_mesh,
    compiler_params=pltpu.CompilerParams(needs_layout_passes=False),
    scratch_types=[
        pltpu.VMEM((C,), jnp.float32),
        pltpu.VMEM((C,), jnp.float32),
        pltpu.VMEM((C,), jnp.float32),
        pltpu.VMEM((C,), jnp.int32),
        pltpu.VMEM((C,), jnp.bool_),
        pltpu.SemaphoreType.DMA,
    ],
)
def _occupancy_kernel(x_hbm, y_hbm, z_hbm, grid_hbm, out_hbm,
                      x_v, y_v, z_v, idx_v, res_v, sem):
    wid = lax.axis_index("s") * NC + lax.axis_index("c")
    n_my_chunks = (N_CHUNKS - wid + NW - 1) // NW

    def chunk_body(i, carry):
        base = (wid + i * NW) * C
        cps = [
            pltpu.async_copy(x_hbm.at[pl.ds(base, C)], x_v, sem),
            pltpu.async_copy(y_hbm.at[pl.ds(base, C)], y_v, sem),
            pltpu.async_copy(z_hbm.at[pl.ds(base, C)], z_v, sem),
        ]
        for cp in cps:
            cp.wait()

        def grp(g, carry2):
            s = pl.ds(g * L, L)
            x = x_v[s]
            y = y_v[s]
            z = z_v[s]
            ix = (x * np.float32(RES)).astype(jnp.int32)
            iy = (y * np.float32(RES)).astype(jnp.int32)
            iz = (z * np.float32(RES)).astype(jnp.int32)
            hi = jnp.maximum(jnp.maximum(x, y), z)
            lo = jnp.minimum(jnp.minimum(x, y), z)
            inv = (hi >= HI) | (lo < LO)
            idx = ix * (RES * RES) + iy * RES + iz
            idx_v[s] = jnp.where(inv, SENTINEL, idx)
            return carry2

        lax.fori_loop(0, GROUPS, grp, 0, unroll=4)

        pltpu.async_copy(grid_hbm.at[idx_v], res_v, sem).wait()
        pltpu.sync_copy(res_v, out_hbm.at[pl.ds(base, C)])
        return carry

    lax.fori_loop(0, n_my_chunks, chunk_body, 0)


def kernel(pts, grid_flat):
    outs = []
    for s in range(N_SEG):
        seg = pts[s * SEG:(s + 1) * SEG]
        x = seg[:, 0]
        y = seg[:, 1]
        z = seg[:, 2]
        outs.append(_occupancy_kernel(x, y, z, grid_flat))
    return jnp.concatenate(outs)
